# baseline (speedup 1.0000x reference)
import jax
import jax.numpy as jnp
from jax import lax
from jax.experimental import pallas as pl
from jax.experimental.pallas import tpu as pltpu


def _make_nn_body(m_total, m_blk):
    def _nn_body(a_ref, k_ref, idx_ref, pooled_s, best_s, bidx_s):
        g = pl.program_id(0)

        @pl.when(g == 0)
        def _():
            a = a_ref[...]
            pooled_s[...] = jnp.sum(a, axis=1) * (1.0 / a.shape[1])

        scores = lax.dot_general(
            pooled_s[...], k_ref[...], (((1,), (1,)), ((), ())),
            preferred_element_type=jnp.float32)
        m = jnp.max(scores, axis=1, keepdims=True)
        col = lax.broadcasted_iota(jnp.int32, scores.shape, 1) + g * m_blk
        am = jnp.min(jnp.where(scores >= m, col, jnp.int32(m_total)),
                     axis=1, keepdims=True)

        @pl.when(g == 0)
        def _():
            best_s[...] = m
            bidx_s[...] = am

        @pl.when(g > 0)
        def _():
            better = m > best_s[...]
            best_s[...] = jnp.where(better, m, best_s[...])
            bidx_s[...] = jnp.where(better, am, bidx_s[...])

        @pl.when(g == pl.num_programs(0) - 1)
        def _():
            idx_ref[...] = bidx_s[...]

    return _nn_body


def _make_fuse_body(height, width):
    hw = height * width

    def _fuse_body(idx_sref, t_ref, u_ref, w_ref, pb_ref, gu_ref, ge_ref,
                   gb_ref, mk_ref, o_ref):
        b = pl.program_id(0)
        lane = idx_sref[b, 0] % 128
        panel = t_ref[...]
        lc = panel.shape[0]
        onehot = lax.broadcasted_iota(jnp.int32, (1, 128), 1) == lane
        cols = [jnp.sum(jnp.where(onehot, panel[c], 0.0), axis=1, keepdims=True)
                for c in range(lc)]
        rf = jnp.transpose(jnp.concatenate(cols, axis=1))

        u = u_ref[0]
        parts = []
        for kh in range(3):
            for kw in range(3):
                dy, dx = kh - 1, kw - 1
                s = dy * width + dx
                if s > 0:
                    sh = jnp.concatenate(
                        [rf[:, s:], jnp.zeros((lc, s), jnp.float32)], axis=1)
                elif s < 0:
                    sh = jnp.concatenate(
                        [jnp.zeros((lc, -s), jnp.float32), rf[:, :hw + s]], axis=1)
                else:
                    sh = rf
                if dx == 0:
                    parts.append(sh)
                else:
                    j = 0 if dx < 0 else 1
                    parts.append(sh * mk_ref[j:j + 1, :])
        patches = jnp.concatenate(parts, axis=0)
        pre = lax.dot_general(
            patches, w_ref[...], (((0,), (0,)), ((), ())),
            preferred_element_type=jnp.float32) + pb_ref[...]
        eng = pre * jax.nn.sigmoid(pre)
        g = (lax.dot_general(u, gu_ref[...], (((1,), (0,)), ((), ())),
                             preferred_element_type=jnp.float32)
             + lax.dot_general(eng, ge_ref[...], (((1,), (0,)), ((), ())),
                               preferred_element_type=jnp.float32)
             + gb_ref[...])
        o_ref[0] = u + jax.nn.sigmoid(g) * eng

    return _fuse_body


def kernel(unet_features, audio_features, mem_keys, mem_values,
           proj_W, proj_b, gate_W, gate_b, conf_W, conf_b):
    B, C, H, W = unet_features.shape
    M, D = mem_keys.shape
    LC = mem_values.shape[1]
    HW = H * W

    MBLK = 1000
    idx2 = pl.pallas_call(
        _make_nn_body(M, MBLK),
        grid=(M // MBLK,),
        in_specs=[
            pl.BlockSpec((B, audio_features.shape[1], D), lambda g: (0, 0, 0)),
            pl.BlockSpec((MBLK, D), lambda g: (g, 0)),
        ],
        out_specs=pl.BlockSpec((B, 1), lambda g: (0, 0)),
        out_shape=jax.ShapeDtypeStruct((B, 1), jnp.int32),
        scratch_shapes=[
            pltpu.VMEM((B, D), jnp.float32),
            pltpu.VMEM((B, 1), jnp.float32),
            pltpu.VMEM((B, 1), jnp.int32),
        ],
    )(audio_features, mem_keys)

    table_p = mem_values.transpose(1, 2, 3, 0).reshape(LC, HW, M)
    wmat = proj_W.transpose(2, 3, 1, 0).reshape(9 * LC, C)
    pb = proj_b.reshape(1, C)
    gu = gate_W[0, :C, 0, 0].reshape(C, 1)
    ge = gate_W[0, C:, 0, 0].reshape(C, 1)
    gb = gate_b.reshape(1, 1)
    unet_r = unet_features.transpose(0, 2, 3, 1).reshape(B, HW, C)
    xcoord = jnp.arange(HW, dtype=jnp.int32) % W
    masks = jnp.stack([(xcoord >= 1).astype(jnp.float32),
                       (xcoord < W - 1).astype(jnp.float32)], axis=0)

    grid_spec = pltpu.PrefetchScalarGridSpec(
        num_scalar_prefetch=1,
        grid=(B,),
        in_specs=[
            pl.BlockSpec((LC, HW, 128),
                         lambda b, idx_ref: (0, 0, idx_ref[b, 0] // 128)),
            pl.BlockSpec((1, HW, C), lambda b, idx_ref: (b, 0, 0)),
            pl.BlockSpec((9 * LC, C), lambda b, idx_ref: (0, 0)),
            pl.BlockSpec((1, C), lambda b, idx_ref: (0, 0)),
            pl.BlockSpec((C, 1), lambda b, idx_ref: (0, 0)),
            pl.BlockSpec((C, 1), lambda b, idx_ref: (0, 0)),
            pl.BlockSpec((1, 1), lambda b, idx_ref: (0, 0)),
            pl.BlockSpec((2, HW), lambda b, idx_ref: (0, 0)),
        ],
        out_specs=pl.BlockSpec((1, HW, C), lambda b, idx_ref: (b, 0, 0)),
    )
    out = pl.pallas_call(
        _make_fuse_body(H, W),
        grid_spec=grid_spec,
        out_shape=jax.ShapeDtypeStruct((B, HW, C), jnp.float32),
    )(idx2, table_p, unet_r, wmat, pb, gu, ge, gb, masks)

    return out.reshape(B, H, W, C).transpose(0, 3, 1, 2)


# speedup vs baseline: 1.0310x; 1.0310x over previous
import jax
import jax.numpy as jnp
from jax import lax
from jax.experimental import pallas as pl
from jax.experimental.pallas import tpu as pltpu


def _nn_body(a_ref, k_ref, idx_ref):
    a = a_ref[...]
    pooled = jnp.sum(a, axis=1) * (1.0 / a.shape[1])
    scores = lax.dot_general(
        pooled, k_ref[...], (((1,), (1,)), ((), ())),
        preferred_element_type=jnp.float32)
    m = jnp.max(scores, axis=1, keepdims=True)
    col = lax.broadcasted_iota(jnp.int32, scores.shape, 1)
    big = jnp.int32(scores.shape[1])
    idx_ref[...] = jnp.min(jnp.where(scores >= m, col, big), axis=1)


def _make_fuse_body(height, width):
    hw = height * width

    def _fuse_body(idx_sref, t_ref, u_ref, w_ref, pb_ref, gu_ref, ge_ref,
                   gb_ref, mk_ref, o_ref):
        b = pl.program_id(0)
        lane = idx_sref[b] % 128
        panel = t_ref[...]
        lc = panel.shape[0]
        onehot = lax.broadcasted_iota(jnp.int32, (1, 128), 1) == lane
        cols = [jnp.sum(jnp.where(onehot, panel[c], 0.0), axis=1, keepdims=True)
                for c in range(lc)]
        rf = jnp.transpose(jnp.concatenate(cols, axis=1))

        u = u_ref[0]
        parts = []
        for kh in range(3):
            for kw in range(3):
                dy, dx = kh - 1, kw - 1
                s = dy * width + dx
                if s > 0:
                    sh = jnp.concatenate(
                        [rf[:, s:], jnp.zeros((lc, s), jnp.float32)], axis=1)
                elif s < 0:
                    sh = jnp.concatenate(
                        [jnp.zeros((lc, -s), jnp.float32), rf[:, :hw + s]], axis=1)
                else:
                    sh = rf
                if dx == 0:
                    parts.append(sh)
                else:
                    j = 0 if dx < 0 else 1
                    parts.append(sh * mk_ref[j:j + 1, :])
        patches = jnp.concatenate(parts, axis=0)
        pre = lax.dot_general(
            patches, w_ref[...], (((0,), (0,)), ((), ())),
            preferred_element_type=jnp.float32) + pb_ref[...]
        eng = pre * jax.nn.sigmoid(pre)
        g = (lax.dot_general(u, gu_ref[...], (((1,), (0,)), ((), ())),
                             preferred_element_type=jnp.float32)
             + lax.dot_general(eng, ge_ref[...], (((1,), (0,)), ((), ())),
                               preferred_element_type=jnp.float32)
             + gb_ref[...])
        o_ref[0] = u + jax.nn.sigmoid(g) * eng

    return _fuse_body


def kernel(unet_features, audio_features, mem_keys, mem_values,
           proj_W, proj_b, gate_W, gate_b, conf_W, conf_b):
    B, C, H, W = unet_features.shape
    M, D = mem_keys.shape
    LC = mem_values.shape[1]
    HW = H * W

    idx = pl.pallas_call(
        _nn_body,
        out_shape=jax.ShapeDtypeStruct((B,), jnp.int32),
    )(audio_features, mem_keys)

    table_p = mem_values.transpose(1, 2, 3, 0).reshape(LC, HW, M)
    wmat = proj_W.transpose(2, 3, 1, 0).reshape(9 * LC, C)
    pb = proj_b.reshape(1, C)
    gu = gate_W[0, :C, 0, 0].reshape(C, 1)
    ge = gate_W[0, C:, 0, 0].reshape(C, 1)
    gb = gate_b.reshape(1, 1)
    unet_r = unet_features.transpose(0, 2, 3, 1).reshape(B, HW, C)
    xcoord = jnp.arange(HW, dtype=jnp.int32) % W
    masks = jnp.stack([(xcoord >= 1).astype(jnp.float32),
                       (xcoord < W - 1).astype(jnp.float32)], axis=0)

    grid_spec = pltpu.PrefetchScalarGridSpec(
        num_scalar_prefetch=1,
        grid=(B,),
        in_specs=[
            pl.BlockSpec((LC, HW, 128),
                         lambda b, idx_ref: (0, 0, idx_ref[b] // 128)),
            pl.BlockSpec((1, HW, C), lambda b, idx_ref: (b, 0, 0)),
            pl.BlockSpec((9 * LC, C), lambda b, idx_ref: (0, 0)),
            pl.BlockSpec((1, C), lambda b, idx_ref: (0, 0)),
            pl.BlockSpec((C, 1), lambda b, idx_ref: (0, 0)),
            pl.BlockSpec((C, 1), lambda b, idx_ref: (0, 0)),
            pl.BlockSpec((1, 1), lambda b, idx_ref: (0, 0)),
            pl.BlockSpec((2, HW), lambda b, idx_ref: (0, 0)),
        ],
        out_specs=pl.BlockSpec((1, HW, C), lambda b, idx_ref: (b, 0, 0)),
    )
    out = pl.pallas_call(
        _make_fuse_body(H, W),
        grid_spec=grid_spec,
        out_shape=jax.ShapeDtypeStruct((B, HW, C), jnp.float32),
    )(idx, table_p, unet_r, wmat, pb, gu, ge, gb, masks)

    return out.reshape(B, H, W, C).transpose(0, 3, 1, 2)
